# baseline (speedup 1.0000x reference)
import functools

import jax
import jax.numpy as jnp
from jax import lax
from jax.experimental import pallas as pl
from jax.experimental.pallas import tpu as pltpu
from jax.experimental.pallas import tpu_sc as plsc

D = 64
NC, NS = 2, 16
NW = NC * NS
LANE = 128
SEQ = 200
NB = 2


def _gather_body(table_hbm, idx_hbm, out_hbm, idx_v, g_buf, t_buf,
                 gsem0, gsem1, ssem0, ssem1):
    c = lax.axis_index("c")
    s = lax.axis_index("s")
    wid = s * NC + c
    gsems = (gsem0, gsem1)
    ssems = (ssem0, ssem1)

    pltpu.sync_copy(idx_hbm.at[wid], idx_v)

    row_vecs = [lax.iota(jnp.int32, 16) + s0 for s0 in range(0, LANE, 16)]

    def fire_gather(p, b):
        return pltpu.async_copy(table_hbm.at[idx_v.at[p]], g_buf.at[b],
                                gsems[b])

    def transpose(b):
        for d in range(D):
            col = jnp.full((16,), d, jnp.int32)
            for k in range(LANE // 16):
                vals = plsc.load_gather(g_buf.at[b], [row_vecs[k], col])
                t_buf[b, d // 8, d % 8, pl.ds(k * 16, 16)] = vals

    fire_gather(0, 0)

    @pl.loop(0, SEQ, step=NB)
    def outer(p):
        for b in range(NB):
            pp = p + b

            @pl.when(pp + 1 < SEQ)
            def _():
                fire_gather(pp + 1, (b + 1) % NB)

            pltpu.make_async_copy(table_hbm.at[idx_v.at[pp]], g_buf.at[b],
                                  gsems[b]).wait()

            @pl.when(pp >= NB)
            def _():
                for tr in range(8):
                    pltpu.make_async_copy(
                        t_buf.at[b].at[tr],
                        out_hbm.at[pp - NB].at[tr].at[wid],
                        ssems[b],
                    ).wait()

            transpose(b)

            for tr in range(8):
                pltpu.async_copy(t_buf.at[b].at[tr],
                                 out_hbm.at[pp].at[tr].at[wid], ssems[b])

    for b in range(NB):
        last = SEQ - NB + b
        for tr in range(8):
            pltpu.make_async_copy(t_buf.at[b].at[tr],
                                  out_hbm.at[last].at[tr].at[wid],
                                  ssems[b]).wait()


def _impl(input_words, table):
    nsent, seq = input_words.shape
    idx3 = (input_words.astype(jnp.int32)
            .T.reshape(seq, NW, LANE).transpose(1, 0, 2))

    mesh = plsc.VectorSubcoreMesh(core_axis_name="c", subcore_axis_name="s")
    out5 = pl.kernel(
        _gather_body,
        out_type=jax.ShapeDtypeStruct((seq, 8, NW, 8, LANE), jnp.float32),
        mesh=mesh,
        scratch_types=[
            pltpu.VMEM((SEQ, LANE), jnp.int32),
            pltpu.VMEM((NB, LANE, D), jnp.float32),
            pltpu.VMEM((NB, 8, 8, LANE), jnp.float32),
            pltpu.SemaphoreType.DMA,
            pltpu.SemaphoreType.DMA,
            pltpu.SemaphoreType.DMA,
            pltpu.SemaphoreType.DMA,
        ],
        compiler_params=pltpu.CompilerParams(use_tc_tiling_on_sc=False,
                                             needs_layout_passes=False),
    )(table, idx3)
    return out5.transpose(2, 4, 0, 1, 3).reshape(nsent, seq, D)


kernel = jax.jit(_impl)


# speedup vs baseline: 1.8241x; 1.8241x over previous
import functools

import jax
import jax.numpy as jnp
from jax import lax
from jax.experimental import pallas as pl
from jax.experimental.pallas import tpu as pltpu
from jax.experimental.pallas import tpu_sc as plsc

D = 64
NC, NS = 2, 16
NW = NC * NS
LANE = 128
SEQ = 200
NB = 2


def _gather_body(table_hbm, idx_hbm, out_hbm, idx_v, g_buf, t_buf,
                 gsem0, gsem1, ssem0, ssem1):
    c = lax.axis_index("c")
    s = lax.axis_index("s")
    wid = s * NC + c
    gsems = (gsem0, gsem1)
    ssems = (ssem0, ssem1)

    pltpu.sync_copy(idx_hbm.at[wid], idx_v)

    row_vecs = [lax.iota(jnp.int32, 16) + d0 for d0 in range(0, D, 16)]
    cols = [jnp.full((16,), l, jnp.int32) for l in range(LANE)]

    def fire_gather(p, b):
        return pltpu.async_copy(table_hbm.at[idx_v.at[p]], g_buf.at[b],
                                gsems[b])

    def transpose(b):
        for l in range(LANE):
            vals = [g_buf[b, l, pl.ds(d0, 16)] for d0 in range(0, D, 16)]
            for ki in range(D // 16):
                plsc.store_scatter(t_buf.at[b], [row_vecs[ki], cols[l]],
                                   vals[ki])

    fire_gather(0, 0)

    @pl.loop(0, SEQ, step=NB)
    def outer(p):
        for b in range(NB):
            pp = p + b

            @pl.when(pp + 1 < SEQ)
            def _():
                fire_gather(pp + 1, (b + 1) % NB)

            pltpu.make_async_copy(table_hbm.at[idx_v.at[pp]], g_buf.at[b],
                                  gsems[b]).wait()

            @pl.when(pp >= NB)
            def _():
                for tr in range(8):
                    pltpu.make_async_copy(
                        t_buf.at[b].at[pl.ds(tr * 8, 8), pl.ds(0, LANE)],
                        out_hbm.at[pp - NB].at[tr].at[wid],
                        ssems[b],
                    ).wait()

            transpose(b)

            for tr in range(8):
                pltpu.async_copy(
                    t_buf.at[b].at[pl.ds(tr * 8, 8), pl.ds(0, LANE)],
                    out_hbm.at[pp].at[tr].at[wid], ssems[b])

    for b in range(NB):
        last = SEQ - NB + b
        for tr in range(8):
            pltpu.make_async_copy(
                t_buf.at[b].at[pl.ds(tr * 8, 8), pl.ds(0, LANE)],
                out_hbm.at[last].at[tr].at[wid], ssems[b]).wait()


def _impl(input_words, table):
    nsent, seq = input_words.shape
    idx3 = (input_words.astype(jnp.int32)
            .T.reshape(seq, NW, LANE).transpose(1, 0, 2))

    mesh = plsc.VectorSubcoreMesh(core_axis_name="c", subcore_axis_name="s")
    out5 = pl.kernel(
        _gather_body,
        out_type=jax.ShapeDtypeStruct((seq, 8, NW, 8, LANE), jnp.float32),
        mesh=mesh,
        scratch_types=[
            pltpu.VMEM((SEQ, LANE), jnp.int32),
            pltpu.VMEM((NB, LANE, D), jnp.float32),
            pltpu.VMEM((NB, D, LANE + 1), jnp.float32),
            pltpu.SemaphoreType.DMA,
            pltpu.SemaphoreType.DMA,
            pltpu.SemaphoreType.DMA,
            pltpu.SemaphoreType.DMA,
        ],
        compiler_params=pltpu.CompilerParams(use_tc_tiling_on_sc=False,
                                             needs_layout_passes=False),
    )(table, idx3)
    return out5.transpose(2, 4, 0, 1, 3).reshape(nsent, seq, D)


kernel = jax.jit(_impl)


# speedup vs baseline: 2.6262x; 1.4397x over previous
import functools

import jax
import jax.numpy as jnp
from jax import lax
from jax.experimental import pallas as pl
from jax.experimental.pallas import tpu as pltpu
from jax.experimental.pallas import tpu_sc as plsc

D = 64
NC, NS = 2, 16
NW = NC * NS
LANE = 128
SEQ = 200
NB = 2


def _gather_body(table_hbm, idx_hbm, out_hbm, idx_v, g_buf, t_buf,
                 gsem0, gsem1, ssem0, ssem1):
    c = lax.axis_index("c")
    s = lax.axis_index("s")
    wid = s * NC + c
    gsems = (gsem0, gsem1)
    ssems = (ssem0, ssem1)

    pltpu.sync_copy(idx_hbm.at[wid], idx_v)

    row_vecs = [lax.iota(jnp.int32, 16) + d0 for d0 in range(0, D, 16)]
    cols = [jnp.full((16,), l, jnp.int32) for l in range(LANE)]

    def fire_gather(p, b):
        return pltpu.async_copy(table_hbm.at[idx_v.at[p]], g_buf.at[b],
                                gsems[b])

    def transpose(b):
        @functools.partial(plsc.parallel_loop, 0, LANE, unroll=8)
        def _(l):
            col = jnp.full((16,), 0, jnp.int32) + l
            vals = [g_buf[b, l, pl.ds(d0, 16)] for d0 in range(0, D, 16)]
            for ki in range(D // 16):
                plsc.store_scatter(t_buf.at[b], [row_vecs[ki], col],
                                   vals[ki])

    fire_gather(0, 0)

    @pl.loop(0, SEQ, step=NB)
    def outer(p):
        for b in range(NB):
            pp = p + b

            @pl.when(pp + 1 < SEQ)
            def _():
                fire_gather(pp + 1, (b + 1) % NB)

            pltpu.make_async_copy(table_hbm.at[idx_v.at[pp]], g_buf.at[b],
                                  gsems[b]).wait()

            @pl.when(pp >= NB)
            def _():
                for tr in range(8):
                    pltpu.make_async_copy(
                        t_buf.at[b].at[pl.ds(tr * 8, 8), pl.ds(0, LANE)],
                        out_hbm.at[pp - NB].at[tr].at[wid],
                        ssems[b],
                    ).wait()

            transpose(b)

            for tr in range(8):
                pltpu.async_copy(
                    t_buf.at[b].at[pl.ds(tr * 8, 8), pl.ds(0, LANE)],
                    out_hbm.at[pp].at[tr].at[wid], ssems[b])

    for b in range(NB):
        last = SEQ - NB + b
        for tr in range(8):
            pltpu.make_async_copy(
                t_buf.at[b].at[pl.ds(tr * 8, 8), pl.ds(0, LANE)],
                out_hbm.at[last].at[tr].at[wid], ssems[b]).wait()


def _impl(input_words, table):
    nsent, seq = input_words.shape
    idx3 = (input_words.astype(jnp.int32)
            .T.reshape(seq, NW, LANE).transpose(1, 0, 2))

    mesh = plsc.VectorSubcoreMesh(core_axis_name="c", subcore_axis_name="s")
    out5 = pl.kernel(
        _gather_body,
        out_type=jax.ShapeDtypeStruct((seq, 8, NW, 8, LANE), jnp.float32),
        mesh=mesh,
        scratch_types=[
            pltpu.VMEM((SEQ, LANE), jnp.int32),
            pltpu.VMEM((NB, LANE, D), jnp.float32),
            pltpu.VMEM((NB, D, LANE + 1), jnp.float32),
            pltpu.SemaphoreType.DMA,
            pltpu.SemaphoreType.DMA,
            pltpu.SemaphoreType.DMA,
            pltpu.SemaphoreType.DMA,
        ],
        compiler_params=pltpu.CompilerParams(use_tc_tiling_on_sc=False,
                                             needs_layout_passes=False),
    )(table, idx3)
    return out5.transpose(2, 4, 0, 1, 3).reshape(nsent, seq, D)


kernel = jax.jit(_impl)
